# baseline (speedup 1.0000x reference)
import jax
import jax.numpy as jnp
from jax import lax
from jax.experimental import pallas as pl
from jax.experimental.pallas import tpu as pltpu
from jax.experimental.pallas import tpu_sc as plsc

NC = 2
NS = 16
NW = NC * NS

CHUNK = 128
K = 1
GROUP = CHUNK * K
NBUF = 6


def _embed_body(idx_hbm, table_hbm, out_hbm, idx_v, rows_v, gsem, wsem):
    nchunks = idx_hbm.shape[0] // NW
    ngroups = nchunks // K
    wid = lax.axis_index("s") * NC + lax.axis_index("c")
    cbase = wid * nchunks
    rbase = cbase * CHUNK

    pltpu.sync_copy(idx_hbm.at[pl.ds(cbase, nchunks)], idx_v)

    def g_copy(g, b, k):
        return (table_hbm.at[idx_v.at[g * K + k]],
                rows_v.at[pl.ds(b * GROUP + k * CHUNK, CHUNK)],
                gsem.at[b])

    def w_copy(g, b):
        return (rows_v.at[pl.ds(b * GROUP, GROUP)],
                out_hbm.at[pl.ds(rbase + g * GROUP, GROUP)],
                wsem.at[b])

    def fire_g(g, b):
        for k in range(K):
            pltpu.async_copy(*g_copy(g, b, k))

    def wait_g(g, b):
        for k in range(K):
            pltpu.make_async_copy(*g_copy(g, b, k)).wait()

    def fire_w(g, b):
        pltpu.async_copy(*w_copy(g, b))

    def wait_w(g, b):
        pltpu.make_async_copy(*w_copy(g, b)).wait()

    for j in range(NBUF - 1):
        fire_g(j, j)
    wait_g(0, 0)
    fire_w(0, 0)
    fire_g(NBUF - 1, NBUF - 1)

    def body(g):
        b = g % NBUF
        wait_g(g, b)
        fire_w(g, b)
        b2 = (g + NBUF - 1) % NBUF
        wait_w(g - 1, b2)
        fire_g(g + NBUF - 1, b2)

    pl.loop(1, ngroups - NBUF + 1)(body)

    for g in range(ngroups - NBUF + 1, ngroups):
        b = g % NBUF
        wait_g(g, b)
        fire_w(g, b)
    for g in range(ngroups - NBUF, ngroups):
        wait_w(g, g % NBUF)


def kernel(x, table):
    B, H = x.shape
    V, D = table.shape
    n = B * H
    idx2d = x.T.reshape(n // CHUNK, CHUNK)

    run = pl.kernel(
        _embed_body,
        out_type=jax.ShapeDtypeStruct((n, D), table.dtype),
        mesh=plsc.VectorSubcoreMesh(core_axis_name="c", subcore_axis_name="s"),
        scratch_types=[
            pltpu.VMEM((n // CHUNK // NW, CHUNK), jnp.int32),
            pltpu.VMEM((NBUF * GROUP, D), jnp.float32),
            pltpu.SemaphoreType.DMA((NBUF,)),
            pltpu.SemaphoreType.DMA((NBUF,)),
        ],
        compiler_params=pltpu.CompilerParams(use_tc_tiling_on_sc=True),
    )
    out = run(idx2d, table)
    return out.reshape(H, B, D).transpose(1, 0, 2)


# speedup vs baseline: 1.0008x; 1.0008x over previous
import jax
import jax.numpy as jnp
from jax import lax
from jax.experimental import pallas as pl
from jax.experimental.pallas import tpu as pltpu
from jax.experimental.pallas import tpu_sc as plsc

NC = 2
NS = 16
NW = NC * NS

CHUNK = 128
K = 1
GROUP = CHUNK * K
NBUF = 6


def _embed_body(idx_hbm, table_hbm, out_hbm, idx_v, rows_v, gsem, wsem):
    nchunks = idx_hbm.shape[0] // NW
    ngroups = nchunks // K
    wid = lax.axis_index("s") * NC + lax.axis_index("c")
    cbase = wid * nchunks
    rbase = cbase * CHUNK

    pltpu.sync_copy(idx_hbm.at[pl.ds(cbase, nchunks)], idx_v)

    def g_copy(g, b, k):
        return (table_hbm.at[idx_v.at[g * K + k]],
                rows_v.at[pl.ds(b * GROUP + k * CHUNK, CHUNK)],
                gsem.at[b])

    def w_copy(g, b):
        return (rows_v.at[pl.ds(b * GROUP, GROUP)],
                out_hbm.at[pl.ds(rbase + g * GROUP, GROUP)],
                wsem.at[b])

    def fire_g(g, b):
        for k in range(K):
            pltpu.async_copy(*g_copy(g, b, k))

    def wait_g(g, b):
        for k in range(K):
            pltpu.make_async_copy(*g_copy(g, b, k)).wait()

    def fire_w(g, b):
        pltpu.async_copy(*w_copy(g, b))

    def wait_w(g, b):
        pltpu.make_async_copy(*w_copy(g, b)).wait()

    for j in range(NBUF - 1):
        fire_g(j, j)
    wait_g(0, 0)
    fire_w(0, 0)
    fire_g(NBUF - 1, NBUF - 1)

    def body(g):
        b = g % NBUF
        wait_g(g, b)
        fire_w(g, b)
        b2 = (g + NBUF - 1) % NBUF
        wait_w(g - 1, b2)
        fire_g(g + NBUF - 1, b2)

    pl.loop(1, ngroups - NBUF + 1)(body)

    for g in range(ngroups - NBUF + 1, ngroups):
        b = g % NBUF
        wait_g(g, b)
        fire_w(g, b)
    for g in range(ngroups - NBUF, ngroups):
        wait_w(g, g % NBUF)


def kernel(x, table):
    B, H = x.shape
    V, D = table.shape
    n = B * H
    idx2d = x.T.reshape(n // CHUNK, CHUNK)

    run = pl.kernel(
        _embed_body,
        out_type=jax.ShapeDtypeStruct((n, D), table.dtype),
        mesh=plsc.VectorSubcoreMesh(core_axis_name="c", subcore_axis_name="s"),
        scratch_types=[
            pltpu.VMEM((n // CHUNK // NW, CHUNK), jnp.int32),
            pltpu.VMEM((NBUF * GROUP, D), jnp.float32),
            pltpu.SemaphoreType.DMA((NBUF,)),
            pltpu.SemaphoreType.DMA((NBUF,)),
        ],
    )
    out = run(idx2d, table)
    return out.reshape(H, B, D).transpose(1, 0, 2)
